# baseline (speedup 1.0000x reference)
import jax
import jax.numpy as jnp
from jax import lax
from jax.experimental import pallas as pl
from jax.experimental.pallas import tpu as pltpu
from jax.experimental.pallas import tpu_sc as plsc

N_IN = 9_633_792
OUT_LEN = 38_535_168
NC = 2
NS = 16
LANES = 16

SZ = 1_835_008
NB = 21
NB0 = 11
ACC_LEN = SZ + 64

W = 1_024
NBUF = 4
PER_TILE = N_IN // NS
NWIN = PER_TILE // W
NVREG = W // LANES

CHUNK = 128
RING = 4
STAGE = W + 2 * CHUNK

PER_TILE_ACC = SZ // NS
NZERO = PER_TILE_ACC // W
FLUSH = PER_TILE_ACC // 4


def _sc_body(upd_hbm, idx_hbm, out_hbm,
             idxwin, updwin, stage_i, stage_u, chunk_i, chunk_u,
             acc, sem_in, sem_z, sem_sc):
  c = lax.axis_index("c")
  s = lax.axis_index("s")

  zero16f = jnp.zeros((LANES,), jnp.float32)
  dump16 = jnp.int32(SZ) + lax.iota(jnp.int32, LANES)
  nbuckets = jnp.where(c == 0, NB0, NB - NB0)

  def wait_chunk():
    pltpu.make_async_copy(chunk_u.at[0], acc.at[chunk_i.at[0]], sem_sc).wait()

  def fire_chunk(src_off, tot):
    slot = lax.rem(tot, RING)

    @pl.when(tot >= RING)
    def _():
      wait_chunk()

    for r in range(CHUNK // LANES):
      chunk_i[slot, pl.ds(r * LANES, LANES)] = (
          stage_i[pl.ds(src_off + r * LANES, LANES)])
      chunk_u[slot, pl.ds(r * LANES, LANES)] = (
          stage_u[pl.ds(src_off + r * LANES, LANES)])
    pltpu.async_copy(chunk_u.at[slot], acc.at[chunk_i.at[slot]], sem_sc,
                     add=True)
    return tot + 1

  def fire_win(w, buf):
    off = s * PER_TILE + w * W
    pltpu.async_copy(idx_hbm.at[pl.ds(off, W)], idxwin.at[buf], sem_in)
    pltpu.async_copy(upd_hbm.at[pl.ds(off, W)], updwin.at[buf], sem_in)

  def wait_win(buf):
    pltpu.make_async_copy(idx_hbm.at[pl.ds(0, W)], idxwin.at[buf],
                          sem_in).wait()
    pltpu.make_async_copy(upd_hbm.at[pl.ds(0, W)], updwin.at[buf],
                          sem_in).wait()

  def bucket_body(p, _):
    b = jnp.where(c == 0, p, NB0 + p)
    base = b * jnp.int32(SZ)

    def zsrc(r, _):
      updwin[0, pl.ds(r * LANES, LANES)] = zero16f
      return 0
    lax.fori_loop(0, W // LANES, zsrc, 0)

    def zfire(j, _):
      pltpu.async_copy(updwin.at[0],
                       acc.at[pl.ds(s * PER_TILE_ACC + j * W, W)], sem_z)
      return 0
    lax.fori_loop(0, NZERO, zfire, 0)

    def zwait(j, _):
      pltpu.make_async_copy(updwin.at[0], acc.at[pl.ds(0, W)], sem_z).wait()
      return 0
    lax.fori_loop(0, NZERO, zwait, 0)

    @pl.when(s == 0)
    def _():
      pltpu.sync_copy(updwin.at[0, pl.ds(0, 64)], acc.at[pl.ds(SZ, 64)])

    plsc.subcore_barrier()

    for pw in range(NBUF - 1):
      fire_win(pw, pw)

    def win_body(w, carry):
      cntv, tot = carry
      buf = lax.rem(w, NBUF)

      @pl.when(w + NBUF - 1 < NWIN)
      def _():
        fire_win(w + NBUF - 1, lax.rem(w + NBUF - 1, NBUF))

      wait_win(buf)

      def vloop(k, cntv):
        iv = idxwin[buf, pl.ds(k * LANES, LANES)]
        local = iv - base
        m = (local >= 0) & (local < SZ)
        pc = plsc.all_reduce_population_count(m)
        rank = plsc.cumsum(m.astype(jnp.int32)) - 1
        pos = cntv + rank
        uv = updwin[buf, pl.ds(k * LANES, LANES)]
        plsc.store_scatter(stage_i, [pos], local, mask=m)
        plsc.store_scatter(stage_u, [pos], uv, mask=m)
        return cntv + pc

      cntv = lax.fori_loop(0, NVREG, vloop, cntv, unroll=4)
      cnt = cntv[0]

      nfull = lax.shift_right_logical(cnt, 7)

      def floop(j, tot):
        return fire_chunk(j * CHUNK, tot)
      tot = lax.fori_loop(0, nfull, floop, tot)

      rem_off = nfull * CHUNK
      for r in range(CHUNK // LANES):
        stage_i[pl.ds(r * LANES, LANES)] = (
            stage_i[pl.ds(rem_off + r * LANES, LANES)])
        stage_u[pl.ds(r * LANES, LANES)] = (
            stage_u[pl.ds(rem_off + r * LANES, LANES)])
      return (cntv - nfull * CHUNK, tot)

    cntv, tot = lax.fori_loop(
        0, NWIN, win_body, (jnp.zeros((LANES,), jnp.int32), jnp.int32(0)))
    cnt = cntv[0]

    for r in range(CHUNK // LANES):
      stage_i[pl.ds(cnt + r * LANES, LANES)] = dump16
      stage_u[pl.ds(cnt + r * LANES, LANES)] = zero16f
    tot = fire_chunk(0, tot)

    def drain(i, _):
      wait_chunk()
      return 0
    lax.fori_loop(0, jnp.minimum(tot, RING), drain, 0)

    plsc.subcore_barrier()

    for mm in range(PER_TILE_ACC // FLUSH):
      start = s * PER_TILE_ACC + mm * FLUSH
      pltpu.sync_copy(acc.at[pl.ds(start, FLUSH)],
                      out_hbm.at[pl.ds(base + start, FLUSH)])

    plsc.subcore_barrier()
    return 0

  lax.fori_loop(0, nbuckets, bucket_body, 0)


@jax.jit
def _scatter_add_sc(upd_flat, idx_flat):
  mesh = plsc.VectorSubcoreMesh(core_axis_name="c", subcore_axis_name="s")
  f = pl.kernel(
      _sc_body,
      out_type=jax.ShapeDtypeStruct((OUT_LEN,), jnp.float32),
      mesh=mesh,
      compiler_params=pltpu.CompilerParams(needs_layout_passes=False),
      scratch_types=[
          pltpu.VMEM((NBUF, W), jnp.int32),
          pltpu.VMEM((NBUF, W), jnp.float32),
          pltpu.VMEM((STAGE,), jnp.int32),
          pltpu.VMEM((STAGE,), jnp.float32),
          pltpu.VMEM((RING, CHUNK), jnp.int32),
          pltpu.VMEM((RING, CHUNK), jnp.float32),
          pltpu.VMEM_SHARED((ACC_LEN,), jnp.float32),
          pltpu.SemaphoreType.DMA,
          pltpu.SemaphoreType.DMA,
          pltpu.SemaphoreType.DMA,
      ],
  )
  return f(upd_flat, idx_flat)


def kernel(updates, mask):
  B, H, Wd, C = updates.shape
  out_h, out_w = H * 2, Wd * 2
  upd_flat = updates.reshape(-1)
  idx_flat = mask.reshape(-1).astype(jnp.int32)
  flat = _scatter_add_sc(upd_flat, idx_flat)
  return flat.reshape(-1, out_h, out_w, C)


# speedup vs baseline: 1.0675x; 1.0675x over previous
import jax
import jax.numpy as jnp
from jax import lax
from jax.experimental import pallas as pl
from jax.experimental.pallas import tpu as pltpu
from jax.experimental.pallas import tpu_sc as plsc

N_IN = 9_633_792
OUT_LEN = 38_535_168
NC = 2
NS = 16
LANES = 16

SZ = 1_835_008
NB = 21
NB0 = 11
ACC_LEN = SZ + 1024

W = 1_024
PER_TILE = N_IN // NS
NWIN = PER_TILE // W
NVREG = W // LANES

CHUNK = 1_024
CROWS = 1
RING = 2
STAGE = 2 * CHUNK

PER_TILE_ACC = SZ // NS
NZERO = PER_TILE_ACC // W
FLUSH = PER_TILE_ACC // 4


def _sc_body(upd_hbm, idx_hbm, out_hbm,
             idxwin, updwin, stage_i, stage_u,
             chunk_i0, chunk_u0, chunk_i1, chunk_u1,
             acc, sem_in, sem_z, sem_sc):
  c = lax.axis_index("c")
  s = lax.axis_index("s")

  zero16f = jnp.zeros((LANES,), jnp.float32)
  dump16 = jnp.int32(SZ) + lax.iota(jnp.int32, LANES)
  nbuckets = jnp.where(c == 0, NB0, NB - NB0)

  def wait_chunk():
    pltpu.make_async_copy(chunk_u0, acc.at[chunk_i0], sem_sc).wait()

  def fire_into(src_off, ci, cu):
    for r in range(CHUNK // LANES):
      ci[pl.ds(r * LANES, LANES)] = stage_i[pl.ds(src_off + r * LANES, LANES)]
      cu[pl.ds(r * LANES, LANES)] = stage_u[pl.ds(src_off + r * LANES, LANES)]
    pltpu.async_copy(cu, acc.at[ci], sem_sc, add=True)

  def fire_chunk(src_off, tot):
    @pl.when(tot >= RING)
    def _():
      wait_chunk()

    @pl.when(lax.rem(tot, RING) == 0)
    def _():
      fire_into(src_off, chunk_i0, chunk_u0)

    @pl.when(lax.rem(tot, RING) == 1)
    def _():
      fire_into(src_off, chunk_i1, chunk_u1)

    return tot + 1

  def fire_win(w, buf):
    off = s * PER_TILE + w * W
    pltpu.async_copy(idx_hbm.at[pl.ds(off, W)], idxwin.at[buf], sem_in)
    pltpu.async_copy(upd_hbm.at[pl.ds(off, W)], updwin.at[buf], sem_in)

  def wait_win(buf):
    pltpu.make_async_copy(idx_hbm.at[pl.ds(0, W)], idxwin.at[buf],
                          sem_in).wait()
    pltpu.make_async_copy(upd_hbm.at[pl.ds(0, W)], updwin.at[buf],
                          sem_in).wait()

  def bucket_body(p, _):
    b = jnp.where(c == 0, p, NB0 + p)
    base = b * jnp.int32(SZ)

    def zsrc(r, _):
      updwin[0, pl.ds(r * LANES, LANES)] = zero16f
      return 0
    lax.fori_loop(0, W // LANES, zsrc, 0)

    def zfire(j, _):
      pltpu.async_copy(updwin.at[0],
                       acc.at[pl.ds(s * PER_TILE_ACC + j * W, W)], sem_z)
      return 0
    lax.fori_loop(0, NZERO, zfire, 0)

    def zwait(j, _):
      pltpu.make_async_copy(updwin.at[0], acc.at[pl.ds(0, W)], sem_z).wait()
      return 0
    lax.fori_loop(0, NZERO, zwait, 0)

    @pl.when(s == 0)
    def _():
      pltpu.sync_copy(updwin.at[0], acc.at[pl.ds(SZ, 1024)])

    plsc.subcore_barrier()

    fire_win(0, 0)

    def win_body(w, carry):
      cntv, tot = carry
      buf = lax.rem(w, 2)

      @pl.when(w + 1 < NWIN)
      def _():
        fire_win(w + 1, 1 - buf)

      wait_win(buf)

      def vloop(k, cntv):
        iv = idxwin[buf, pl.ds(k * LANES, LANES)]
        local = iv - base
        m = (local >= 0) & (local < SZ)
        pc = plsc.all_reduce_population_count(m)
        rank = plsc.cumsum(m.astype(jnp.int32)) - 1
        pos = cntv + rank
        uv = updwin[buf, pl.ds(k * LANES, LANES)]
        plsc.store_scatter(stage_i, [pos], local, mask=m)
        plsc.store_scatter(stage_u, [pos], uv, mask=m)
        return cntv + pc

      cntv = lax.fori_loop(0, NVREG, vloop, cntv, unroll=4)
      cnt = cntv[0]

      nfull = lax.shift_right_logical(cnt, 10)

      def flush_and_shift(tot):
        tot = fire_chunk(0, tot)
        for r in range(CHUNK // LANES):
          stage_i[pl.ds(r * LANES, LANES)] = (
              stage_i[pl.ds(CHUNK + r * LANES, LANES)])
          stage_u[pl.ds(r * LANES, LANES)] = (
              stage_u[pl.ds(CHUNK + r * LANES, LANES)])
        return tot

      tot = lax.cond(nfull > 0, flush_and_shift, lambda t: t, tot)
      return (cntv - nfull * CHUNK, tot)

    cntv, tot = lax.fori_loop(
        0, NWIN, win_body, (jnp.zeros((LANES,), jnp.int32), jnp.int32(0)))
    cnt = cntv[0]

    for r in range(CHUNK // LANES):
      stage_i[pl.ds(cnt + r * LANES, LANES)] = dump16 + (r * LANES)
      stage_u[pl.ds(cnt + r * LANES, LANES)] = zero16f
    tot = fire_chunk(0, tot)

    def drain(i, _):
      wait_chunk()
      return 0
    lax.fori_loop(0, jnp.minimum(tot, RING), drain, 0)

    plsc.subcore_barrier()

    for mm in range(PER_TILE_ACC // FLUSH):
      start = s * PER_TILE_ACC + mm * FLUSH
      pltpu.sync_copy(acc.at[pl.ds(start, FLUSH)],
                      out_hbm.at[pl.ds(base + start, FLUSH)])

    plsc.subcore_barrier()
    return 0

  lax.fori_loop(0, nbuckets, bucket_body, 0)


@jax.jit
def _scatter_add_sc(upd_flat, idx_flat):
  mesh = plsc.VectorSubcoreMesh(core_axis_name="c", subcore_axis_name="s")
  f = pl.kernel(
      _sc_body,
      out_type=jax.ShapeDtypeStruct((OUT_LEN,), jnp.float32),
      mesh=mesh,
      compiler_params=pltpu.CompilerParams(needs_layout_passes=False),
      scratch_types=[
          pltpu.VMEM((2, W), jnp.int32),
          pltpu.VMEM((2, W), jnp.float32),
          pltpu.VMEM((STAGE,), jnp.int32),
          pltpu.VMEM((STAGE,), jnp.float32),
          pltpu.VMEM((CHUNK,), jnp.int32),
          pltpu.VMEM((CHUNK,), jnp.float32),
          pltpu.VMEM((CHUNK,), jnp.int32),
          pltpu.VMEM((CHUNK,), jnp.float32),
          pltpu.VMEM_SHARED((ACC_LEN,), jnp.float32),
          pltpu.SemaphoreType.DMA,
          pltpu.SemaphoreType.DMA,
          pltpu.SemaphoreType.DMA,
      ],
  )
  return f(upd_flat, idx_flat)


def kernel(updates, mask):
  B, H, Wd, C = updates.shape
  out_h, out_w = H * 2, Wd * 2
  upd_flat = updates.reshape(-1)
  idx_flat = mask.reshape(-1).astype(jnp.int32)
  flat = _scatter_add_sc(upd_flat, idx_flat)
  return flat.reshape(-1, out_h, out_w, C)


# speedup vs baseline: 2.1106x; 1.9771x over previous
import jax
import jax.numpy as jnp
from jax import lax
from jax.experimental import pallas as pl
from jax.experimental.pallas import tpu as pltpu
from jax.experimental.pallas import tpu_sc as plsc

N_IN = 9_633_792
OUT_LEN = 38_535_168
NC = 2
NS = 16
LANES = 16

SZ = 1_835_008
NB = 21
NB0 = 11
ACC_LEN = SZ + 1024

W = 1_024
PER_TILE = N_IN // NS
NWIN = PER_TILE // W
NVREG = W // LANES

CHUNK = 1_024
CROWS = 1
RING = 2
STAGE = 2 * CHUNK

PER_TILE_ACC = SZ // NS
NZERO = PER_TILE_ACC // W
FLUSH = PER_TILE_ACC // 4


def _sc_body(upd_hbm, idx_hbm, out_hbm,
             idxwin, updwin, stage_i, stage_u,
             chunk_i0, chunk_u0, chunk_i1, chunk_u1,
             acc, sem_in, sem_z, sem_sc):
  c = lax.axis_index("c")
  s = lax.axis_index("s")

  zero16f = jnp.zeros((LANES,), jnp.float32)
  dump16 = jnp.int32(SZ) + lax.iota(jnp.int32, LANES)
  nbuckets = jnp.where(c == 0, NB0, NB - NB0)

  def wait_chunk():
    pltpu.make_async_copy(chunk_u0, acc.at[chunk_i0], sem_sc).wait()

  def fire_into(src_off, ci, cu):
    for r in range(CHUNK // LANES):
      ci[pl.ds(r * LANES, LANES)] = stage_i[pl.ds(src_off + r * LANES, LANES)]
      cu[pl.ds(r * LANES, LANES)] = stage_u[pl.ds(src_off + r * LANES, LANES)]
    pltpu.async_copy(cu, acc.at[ci], sem_sc, add=True)

  def fire_chunk(src_off, tot):
    @pl.when(tot >= RING)
    def _():
      wait_chunk()

    @pl.when(lax.rem(tot, RING) == 0)
    def _():
      fire_into(src_off, chunk_i0, chunk_u0)

    @pl.when(lax.rem(tot, RING) == 1)
    def _():
      fire_into(src_off, chunk_i1, chunk_u1)

    return tot + 1

  def fire_win(w, buf):
    off = s * PER_TILE + w * W
    pltpu.async_copy(idx_hbm.at[pl.ds(off, W)], idxwin.at[buf], sem_in)
    pltpu.async_copy(upd_hbm.at[pl.ds(off, W)], updwin.at[buf], sem_in)

  def wait_win(buf):
    pltpu.make_async_copy(idx_hbm.at[pl.ds(0, W)], idxwin.at[buf],
                          sem_in).wait()
    pltpu.make_async_copy(upd_hbm.at[pl.ds(0, W)], updwin.at[buf],
                          sem_in).wait()

  def bucket_body(p, _):
    b = jnp.where(c == 0, p, NB0 + p)
    base = b * jnp.int32(SZ)

    def zsrc(r, _):
      updwin[0, pl.ds(r * LANES, LANES)] = zero16f
      return 0
    lax.fori_loop(0, W // LANES, zsrc, 0)

    def zfire(j, _):
      pltpu.async_copy(updwin.at[0],
                       acc.at[pl.ds(s * PER_TILE_ACC + j * W, W)], sem_z)
      return 0
    lax.fori_loop(0, NZERO, zfire, 0)

    def zwait(j, _):
      pltpu.make_async_copy(updwin.at[0], acc.at[pl.ds(0, W)], sem_z).wait()
      return 0
    lax.fori_loop(0, NZERO, zwait, 0)

    @pl.when(s == 0)
    def _():
      pltpu.sync_copy(updwin.at[0], acc.at[pl.ds(SZ, 1024)])

    plsc.subcore_barrier()

    fire_win(0, 0)

    def win_body(w, carry):
      cntv, tot = carry
      buf = lax.rem(w, 2)

      @pl.when(w + 1 < NWIN)
      def _():
        fire_win(w + 1, 1 - buf)

      wait_win(buf)

      def vloop(k, cntv):
        iv = idxwin[buf, pl.ds(k * LANES, LANES)]
        local = iv - base
        m = (local >= 0) & (local < SZ)
        pc = plsc.all_reduce_population_count(m)
        rank = plsc.cumsum(m.astype(jnp.int32)) - 1
        pos = cntv + rank
        uv = updwin[buf, pl.ds(k * LANES, LANES)]
        plsc.store_scatter(stage_i, [pos], local, mask=m)
        plsc.store_scatter(stage_u, [pos], uv, mask=m)
        return cntv + pc

      cntv = plsc.parallel_loop(0, NVREG, unroll=4, carry=cntv)(vloop)
      cnt = cntv[0]

      nfull = lax.shift_right_logical(cnt, 10)

      def flush_and_shift(tot):
        tot = fire_chunk(0, tot)
        for r in range(CHUNK // LANES):
          stage_i[pl.ds(r * LANES, LANES)] = (
              stage_i[pl.ds(CHUNK + r * LANES, LANES)])
          stage_u[pl.ds(r * LANES, LANES)] = (
              stage_u[pl.ds(CHUNK + r * LANES, LANES)])
        return tot

      tot = lax.cond(nfull > 0, flush_and_shift, lambda t: t, tot)
      return (cntv - nfull * CHUNK, tot)

    cntv, tot = lax.fori_loop(
        0, NWIN, win_body, (jnp.zeros((LANES,), jnp.int32), jnp.int32(0)))
    cnt = cntv[0]

    for r in range(CHUNK // LANES):
      stage_i[pl.ds(cnt + r * LANES, LANES)] = dump16 + (r * LANES)
      stage_u[pl.ds(cnt + r * LANES, LANES)] = zero16f
    tot = fire_chunk(0, tot)

    def drain(i, _):
      wait_chunk()
      return 0
    lax.fori_loop(0, jnp.minimum(tot, RING), drain, 0)

    plsc.subcore_barrier()

    for mm in range(PER_TILE_ACC // FLUSH):
      start = s * PER_TILE_ACC + mm * FLUSH
      pltpu.sync_copy(acc.at[pl.ds(start, FLUSH)],
                      out_hbm.at[pl.ds(base + start, FLUSH)])

    plsc.subcore_barrier()
    return 0

  lax.fori_loop(0, nbuckets, bucket_body, 0)


@jax.jit
def _scatter_add_sc(upd_flat, idx_flat):
  mesh = plsc.VectorSubcoreMesh(core_axis_name="c", subcore_axis_name="s")
  f = pl.kernel(
      _sc_body,
      out_type=jax.ShapeDtypeStruct((OUT_LEN,), jnp.float32),
      mesh=mesh,
      compiler_params=pltpu.CompilerParams(needs_layout_passes=False),
      scratch_types=[
          pltpu.VMEM((2, W), jnp.int32),
          pltpu.VMEM((2, W), jnp.float32),
          pltpu.VMEM((STAGE,), jnp.int32),
          pltpu.VMEM((STAGE,), jnp.float32),
          pltpu.VMEM((CHUNK,), jnp.int32),
          pltpu.VMEM((CHUNK,), jnp.float32),
          pltpu.VMEM((CHUNK,), jnp.int32),
          pltpu.VMEM((CHUNK,), jnp.float32),
          pltpu.VMEM_SHARED((ACC_LEN,), jnp.float32),
          pltpu.SemaphoreType.DMA,
          pltpu.SemaphoreType.DMA,
          pltpu.SemaphoreType.DMA,
      ],
  )
  return f(upd_flat, idx_flat)


def kernel(updates, mask):
  B, H, Wd, C = updates.shape
  out_h, out_w = H * 2, Wd * 2
  upd_flat = updates.reshape(-1)
  idx_flat = mask.reshape(-1).astype(jnp.int32)
  flat = _scatter_add_sc(upd_flat, idx_flat)
  return flat.reshape(-1, out_h, out_w, C)


# speedup vs baseline: 2.7788x; 1.3166x over previous
import jax
import jax.numpy as jnp
from jax import lax
from jax.experimental import pallas as pl
from jax.experimental.pallas import tpu as pltpu
from jax.experimental.pallas import tpu_sc as plsc

N_IN = 9_633_792
OUT_LEN = 38_535_168
NC = 2
NS = 16
LANES = 16

SZ = 1_835_008
NB = 21
NB0 = 11
ACC_LEN = SZ + 1024

W = 2_048
HALF = W // 2
PER_TILE = N_IN // NS
NWIN = PER_TILE // W
NVREG = HALF // LANES

CHUNK = 1_024
STAGE = 2 * CHUNK

PER_TILE_ACC = SZ // NS
NZERO = PER_TILE_ACC // W
FLUSH = PER_TILE_ACC // 4


def _sc_body(upd_hbm, idx_hbm, out_hbm,
             idxwin, updwin, stage_i, stage_u,
             acc, sem_in, sem_z):
  c = lax.axis_index("c")
  s = lax.axis_index("s")

  zero16f = jnp.zeros((LANES,), jnp.float32)
  dump16 = jnp.int32(SZ) + lax.iota(jnp.int32, LANES)
  nbuckets = jnp.where(c == 0, NB0, NB - NB0)

  def scatter_head():
    pltpu.sync_copy(stage_u.at[pl.ds(0, CHUNK)],
                    acc.at[stage_i.at[pl.ds(0, CHUNK)]], add=True)

  def shift_tail():
    for r in range(CHUNK // LANES):
      stage_i[pl.ds(r * LANES, LANES)] = (
          stage_i[pl.ds(CHUNK + r * LANES, LANES)])
      stage_u[pl.ds(r * LANES, LANES)] = (
          stage_u[pl.ds(CHUNK + r * LANES, LANES)])

  def fire_win(w, buf):
    off = s * PER_TILE + w * W
    pltpu.async_copy(idx_hbm.at[pl.ds(off, W)], idxwin.at[buf], sem_in)
    pltpu.async_copy(upd_hbm.at[pl.ds(off, W)], updwin.at[buf], sem_in)

  def wait_win(buf):
    pltpu.make_async_copy(idx_hbm.at[pl.ds(0, W)], idxwin.at[buf],
                          sem_in).wait()
    pltpu.make_async_copy(upd_hbm.at[pl.ds(0, W)], updwin.at[buf],
                          sem_in).wait()

  def bucket_body(p, _):
    b = jnp.where(c == 0, p, NB0 + p)
    base = b * jnp.int32(SZ)

    def zsrc(r, _):
      updwin[0, pl.ds(r * LANES, LANES)] = zero16f
      return 0
    lax.fori_loop(0, W // LANES, zsrc, 0)

    def zfire(j, _):
      pltpu.async_copy(updwin.at[0],
                       acc.at[pl.ds(s * PER_TILE_ACC + j * W, W)], sem_z)
      return 0
    lax.fori_loop(0, NZERO, zfire, 0)

    def zwait(j, _):
      pltpu.make_async_copy(updwin.at[0], acc.at[pl.ds(0, W)], sem_z).wait()
      return 0
    lax.fori_loop(0, NZERO, zwait, 0)

    @pl.when(s == 0)
    def _():
      pltpu.sync_copy(updwin.at[0, pl.ds(0, 1024)], acc.at[pl.ds(SZ, 1024)])

    plsc.subcore_barrier()

    fire_win(0, 0)

    def win_body(w, cntv):
      buf = lax.rem(w, 2)

      @pl.when(w + 1 < NWIN)
      def _():
        fire_win(w + 1, 1 - buf)

      wait_win(buf)

      for h in range(W // HALF):
        def vloop(k, cntv):
          iv = idxwin[buf, pl.ds(k * LANES, LANES)]
          local = iv - base
          m = (local >= 0) & (local < SZ)
          pc = plsc.all_reduce_population_count(m)
          rank = plsc.cumsum(m.astype(jnp.int32)) - 1
          pos = cntv + rank
          uv = updwin[buf, pl.ds(k * LANES, LANES)]
          plsc.store_scatter(stage_i, [pos], local, mask=m)
          plsc.store_scatter(stage_u, [pos], uv, mask=m)
          return cntv + pc

        cntv = plsc.parallel_loop(h * NVREG, (h + 1) * NVREG, unroll=4,
                                  carry=cntv)(vloop)
        full = cntv[0] >= CHUNK

        @pl.when(full)
        def _():
          scatter_head()
          shift_tail()

        cntv = jnp.where(full, cntv - CHUNK, cntv)
      return cntv

    cntv = lax.fori_loop(0, NWIN, win_body, jnp.zeros((LANES,), jnp.int32))
    cnt = cntv[0]

    for r in range(CHUNK // LANES):
      stage_i[pl.ds(cnt + r * LANES, LANES)] = dump16 + (r * LANES)
      stage_u[pl.ds(cnt + r * LANES, LANES)] = zero16f
    scatter_head()

    plsc.subcore_barrier()

    for mm in range(PER_TILE_ACC // FLUSH):
      start = s * PER_TILE_ACC + mm * FLUSH
      pltpu.sync_copy(acc.at[pl.ds(start, FLUSH)],
                      out_hbm.at[pl.ds(base + start, FLUSH)])

    plsc.subcore_barrier()
    return 0

  lax.fori_loop(0, nbuckets, bucket_body, 0)


@jax.jit
def _scatter_add_sc(upd_flat, idx_flat):
  mesh = plsc.VectorSubcoreMesh(core_axis_name="c", subcore_axis_name="s")
  f = pl.kernel(
      _sc_body,
      out_type=jax.ShapeDtypeStruct((OUT_LEN,), jnp.float32),
      mesh=mesh,
      compiler_params=pltpu.CompilerParams(needs_layout_passes=False),
      scratch_types=[
          pltpu.VMEM((2, W), jnp.int32),
          pltpu.VMEM((2, W), jnp.float32),
          pltpu.VMEM((STAGE,), jnp.int32),
          pltpu.VMEM((STAGE,), jnp.float32),
          pltpu.VMEM_SHARED((ACC_LEN,), jnp.float32),
          pltpu.SemaphoreType.DMA,
          pltpu.SemaphoreType.DMA,
      ],
  )
  return f(upd_flat, idx_flat)


def kernel(updates, mask):
  B, H, Wd, C = updates.shape
  out_h, out_w = H * 2, Wd * 2
  upd_flat = updates.reshape(-1)
  idx_flat = mask.reshape(-1).astype(jnp.int32)
  flat = _scatter_add_sc(upd_flat, idx_flat)
  return flat.reshape(-1, out_h, out_w, C)
